# baseline (speedup 1.0000x reference)
import functools

import jax
import jax.numpy as jnp
from jax import lax
from jax.experimental import pallas as pl
from jax.experimental.pallas import tpu as pltpu
from jax.experimental.pallas import tpu_sc as plsc

_NC = 2
_NS = 16
_NW = _NC * _NS
_L = 16


def _tc_body(x_ref, p_ref, o_ref):
    o_ref[...] = x_ref[...] + p_ref[...]


def _tc_add(x, pos_embedding, sblk=512):
    B, S, D = x.shape
    grid = (S // sblk, B)
    return pl.pallas_call(
        _tc_body,
        grid=grid,
        in_specs=[
            pl.BlockSpec((1, sblk, D), lambda i, b: (b, i, 0)),
            pl.BlockSpec((sblk, D), lambda i, b: (i, 0)),
        ],
        out_specs=pl.BlockSpec((1, sblk, D), lambda i, b: (b, i, 0)),
        out_shape=jax.ShapeDtypeStruct((B, S, D), x.dtype),
    )(x, pos_embedding)


def _sc_add(x, pos_embedding):
    B, S, D = x.shape
    rows_per_w = S // _NW
    n_chunks = rows_per_w

    mesh = plsc.VectorSubcoreMesh(core_axis_name="c", subcore_axis_name="s")

    @functools.partial(
        pl.kernel,
        out_type=jax.ShapeDtypeStruct((B, S, D), jnp.float32),
        mesh=mesh,
        scratch_types=(
            [pltpu.VMEM((B, 1, D), jnp.float32)] * 4
            + [pltpu.VMEM((1, D), jnp.float32)] * 2
            + [pltpu.SemaphoreType.DMA] * 10
        ),
    )
    def k(x_hbm, pos_hbm, out_hbm, *scr):
        iob = scr[0:4]
        posb = scr[4:6]
        in_sem = scr[6:10]
        out_sem = scr[10:14]
        pos_sem = scr[14:16]

        wid = lax.axis_index("s") * _NC + lax.axis_index("c")
        s_base = wid * rows_per_w

        def in_copy(c, slot):
            return pltpu.make_async_copy(
                x_hbm.at[:, pl.ds(s_base + c, 1), :], iob[slot], in_sem[slot])

        def out_copy(c, slot):
            return pltpu.make_async_copy(
                iob[slot], out_hbm.at[:, pl.ds(s_base + c, 1), :],
                out_sem[slot])

        def pos_copy(c, pc):
            return pltpu.make_async_copy(
                pos_hbm.at[pl.ds(s_base + c, 1)], posb[pc], pos_sem[pc])

        pos_copy(0, 0).start()
        pos_copy(1, 1).start()
        in_copy(0, 0).start()
        in_copy(1, 1).start()

        def period(g, carry):
            for k_slot in range(4):
                pc = k_slot % 2
                c = 4 * g + k_slot
                pos_copy(c, pc).wait()
                in_copy(c, k_slot).wait()

                @plsc.parallel_loop(0, D // _L, unroll=8)
                def _(j):
                    col = j * _L
                    p = posb[pc][0, pl.ds(col, _L)]
                    for b in range(B):
                        plsc.addupdate(
                            iob[k_slot].at[b, 0, pl.ds(col, _L)], p)

                out_copy(c, k_slot).start()

                if k_slot < 2:
                    @pl.when(g > 0)
                    def _():
                        out_copy(4 * g - 2 + k_slot, k_slot + 2).wait()
                    in_copy(4 * g + 2 + k_slot, k_slot + 2).start()
                    pos_copy(4 * g + 2 + k_slot, pc).start()
                else:
                    out_copy(4 * g + k_slot - 2, k_slot - 2).wait()

                    @pl.when(c + 2 < n_chunks)
                    def _():
                        in_copy(4 * g + 2 + k_slot, k_slot - 2).start()
                        pos_copy(4 * g + 2 + k_slot, pc).start()
            return carry

        lax.fori_loop(0, n_chunks // 4, period, 0)

        out_copy(n_chunks - 2, 2).wait()
        out_copy(n_chunks - 1, 3).wait()

    return k(x, pos_embedding)


def kernel(x, pos_embedding):
    return _sc_add(x, pos_embedding)


# speedup vs baseline: 1.0085x; 1.0085x over previous
import functools

import jax
import jax.numpy as jnp
from jax import lax
from jax.experimental import pallas as pl
from jax.experimental.pallas import tpu as pltpu
from jax.experimental.pallas import tpu_sc as plsc

_NC = 2
_NS = 16
_NW = _NC * _NS
_L = 16


def _tc_body(x_ref, p_ref, o_ref):
    o_ref[...] = x_ref[...] + p_ref[...]


def _tc_add(x, pos_embedding, sblk=512):
    B, S, D = x.shape
    grid = (S // sblk, B)
    return pl.pallas_call(
        _tc_body,
        grid=grid,
        in_specs=[
            pl.BlockSpec((1, sblk, D), lambda i, b: (b, i, 0)),
            pl.BlockSpec((sblk, D), lambda i, b: (i, 0)),
        ],
        out_specs=pl.BlockSpec((1, sblk, D), lambda i, b: (b, i, 0)),
        out_shape=jax.ShapeDtypeStruct((B, S, D), x.dtype),
    )(x, pos_embedding)


def _sc_add(x, pos_embedding):
    B, S, D = x.shape
    xf = x.reshape(B * S, D)
    rows_per_w = S // _NW
    R = 2
    n_chunks = rows_per_w // R
    n_iters = n_chunks // 2

    mesh = plsc.VectorSubcoreMesh(core_axis_name="c", subcore_axis_name="s")

    @functools.partial(
        pl.kernel,
        out_type=jax.ShapeDtypeStruct((B * S, D), jnp.float32),
        mesh=mesh,
        scratch_types=(
            [pltpu.VMEM((R, D), jnp.float32)] * 8
            + [pltpu.VMEM((R, D), jnp.float32)] * 2
            + [pltpu.SemaphoreType.DMA] * 18
        ),
    )
    def k(x_hbm, pos_hbm, out_hbm, *scr):
        iob = scr[0:8]
        posb = scr[8:10]
        in_sem = scr[10:18]
        out_sem = scr[18:26]
        pos_sem = scr[26:28]

        wid = lax.axis_index("s") * _NC + lax.axis_index("c")
        s_base = wid * rows_per_w

        def in_copy(c, b, slot):
            row0 = b * S + s_base + c * R
            return pltpu.make_async_copy(
                x_hbm.at[pl.ds(row0, R)], iob[slot], in_sem[slot])

        def out_copy(c, b, slot):
            row0 = b * S + s_base + c * R
            return pltpu.make_async_copy(
                iob[slot], out_hbm.at[pl.ds(row0, R)], out_sem[slot])

        def pos_copy(c, pc):
            return pltpu.make_async_copy(
                pos_hbm.at[pl.ds(s_base + c * R, R)], posb[pc], pos_sem[pc])

        pos_copy(0, 0).start()
        pos_copy(1, 1).start()
        for b in range(B):
            in_copy(0, b, b).start()

        def period(g, carry):
            for k_slot in range(2 * B):
                pc = k_slot // B
                b = k_slot % B
                c = 2 * g + pc
                if b == 0:
                    pos_copy(c, pc).wait()
                in_copy(c, b, k_slot).wait()

                @plsc.parallel_loop(0, D // _L, unroll=8)
                def _(j):
                    col = j * _L
                    for r in range(R):
                        plsc.addupdate(
                            iob[k_slot].at[r, pl.ds(col, _L)],
                            posb[pc][r, pl.ds(col, _L)],
                        )
                out_copy(c, b, k_slot).start()

                if b == B - 1:
                    @pl.when(c + 2 < n_chunks)
                    def _():
                        pos_copy(c + 2, pc).start()

                if k_slot < B:
                    @pl.when(g > 0)
                    def _():
                        out_copy(2 * g - 1, b, k_slot + B).wait()
                    in_copy(2 * g + 1, b, k_slot + B).start()
                else:
                    out_copy(2 * g, b, k_slot - B).wait()

                    @pl.when(g < n_iters - 1)
                    def _():
                        in_copy(2 * g + 2, b, k_slot - B).start()
            return carry

        lax.fori_loop(0, n_iters, period, 0)

        for b in range(B):
            out_copy(n_chunks - 1, b, b + B).wait()

    return k(xf, pos_embedding).reshape(B, S, D)


def kernel(x, pos_embedding):
    return _sc_add(x, pos_embedding)


# speedup vs baseline: 1.0091x; 1.0006x over previous
import functools

import jax
import jax.numpy as jnp
from jax import lax
from jax.experimental import pallas as pl
from jax.experimental.pallas import tpu as pltpu
from jax.experimental.pallas import tpu_sc as plsc

_NC = 2
_NS = 16
_NW = _NC * _NS
_L = 16


def _tc_body(x_ref, p_ref, o_ref):
    o_ref[...] = x_ref[...] + p_ref[...]


def _tc_add(x, pos_embedding, sblk=512):
    B, S, D = x.shape
    grid = (S // sblk, B)
    return pl.pallas_call(
        _tc_body,
        grid=grid,
        in_specs=[
            pl.BlockSpec((1, sblk, D), lambda i, b: (b, i, 0)),
            pl.BlockSpec((sblk, D), lambda i, b: (i, 0)),
        ],
        out_specs=pl.BlockSpec((1, sblk, D), lambda i, b: (b, i, 0)),
        out_shape=jax.ShapeDtypeStruct((B, S, D), x.dtype),
    )(x, pos_embedding)


def _sc_add(x, pos_embedding):
    B, S, D = x.shape
    xf = x.reshape(B * S, D)
    rows_per_w = S // _NW
    R = 2
    n_chunks = rows_per_w // R
    n_iters = n_chunks // 2

    mesh = plsc.VectorSubcoreMesh(core_axis_name="c", subcore_axis_name="s")

    @functools.partial(
        pl.kernel,
        out_type=jax.ShapeDtypeStruct((B * S, D), jnp.float32),
        mesh=mesh,
        scratch_types=(
            [pltpu.VMEM((R, D), jnp.float32)] * 8
            + [pltpu.VMEM((R, D), jnp.float32)] * 2
            + [pltpu.SemaphoreType.DMA] * 18
        ),
    )
    def k(x_hbm, pos_hbm, out_hbm, *scr):
        iob = scr[0:8]
        posb = scr[8:10]
        in_sem = scr[10:18]
        out_sem = scr[18:26]
        pos_sem = scr[26:28]

        wid = lax.axis_index("c") * _NS + lax.axis_index("s")
        s_base = wid * rows_per_w

        def in_copy(c, b, slot):
            row0 = b * S + s_base + c * R
            return pltpu.make_async_copy(
                x_hbm.at[pl.ds(row0, R)], iob[slot], in_sem[slot])

        def out_copy(c, b, slot):
            row0 = b * S + s_base + c * R
            return pltpu.make_async_copy(
                iob[slot], out_hbm.at[pl.ds(row0, R)], out_sem[slot])

        def pos_copy(c, pc):
            return pltpu.make_async_copy(
                pos_hbm.at[pl.ds(s_base + c * R, R)], posb[pc], pos_sem[pc])

        pos_copy(0, 0).start()
        pos_copy(1, 1).start()
        for b in range(B):
            in_copy(0, b, b).start()

        def period(g, carry):
            for k_slot in range(2 * B):
                pc = k_slot // B
                b = k_slot % B
                c = 2 * g + pc
                if b == 0:
                    pos_copy(c, pc).wait()
                in_copy(c, b, k_slot).wait()

                @plsc.parallel_loop(0, D // _L, unroll=8)
                def _(j):
                    col = j * _L
                    for r in range(R):
                        plsc.addupdate(
                            iob[k_slot].at[r, pl.ds(col, _L)],
                            posb[pc][r, pl.ds(col, _L)],
                        )
                out_copy(c, b, k_slot).start()

                if b == B - 1:
                    @pl.when(c + 2 < n_chunks)
                    def _():
                        pos_copy(c + 2, pc).start()

                if k_slot < B:
                    @pl.when(g > 0)
                    def _():
                        out_copy(2 * g - 1, b, k_slot + B).wait()
                    in_copy(2 * g + 1, b, k_slot + B).start()
                else:
                    out_copy(2 * g, b, k_slot - B).wait()

                    @pl.when(g < n_iters - 1)
                    def _():
                        in_copy(2 * g + 2, b, k_slot - B).start()
            return carry

        lax.fori_loop(0, n_iters, period, 0)

        for b in range(B):
            out_copy(n_chunks - 1, b, b + B).wait()

    return k(xf, pos_embedding).reshape(B, S, D)


def kernel(x, pos_embedding):
    return _sc_add(x, pos_embedding)
